# Initial kernel scaffold; baseline (speedup 1.0000x reference)
import jax
import jax.numpy as jnp
from jax.experimental import pallas as pl


def kernel(x, edge_index, W1, b1, W2, b2):
    raise NotImplementedError("write your pallas kernel here")


# speedup vs baseline: 2.9818x; 2.9818x over previous
import functools

import jax
import jax.numpy as jnp
from jax import lax
from jax.experimental import pallas as pl
from jax.experimental.pallas import tpu as pltpu
from jax.experimental.pallas import tpu_sc as plsc

N = 10000
E = 320000
D = 128

NC = 2
NS = 16
NW = NC * NS
CHUNK = 128
CPT = 80
E_PAD = NW * CPT * CHUNK
RPT = 632
NROWS = NS * RPT


def _mm_body(h_ref, w_ref, o_ref):
    o_ref[...] = jnp.dot(h_ref[...], w_ref[...],
                         preferred_element_type=jnp.float32)


def _matmul(h, W, rows_per_block=2000):
    return pl.pallas_call(
        _mm_body,
        grid=(N // rows_per_block,),
        in_specs=[
            pl.BlockSpec((rows_per_block, D), lambda i: (i, 0)),
            pl.BlockSpec((D, D), lambda i: (0, 0)),
        ],
        out_specs=pl.BlockSpec((rows_per_block, D), lambda i: (i, 0)),
        out_shape=jax.ShapeDtypeStruct((N, D), jnp.float32),
    )(h, W)


def _mid_body(p_ref, b_ref, w_ref, o_ref):
    h = jnp.maximum(p_ref[0] + p_ref[1] + b_ref[...], 0.0)
    o_ref[...] = jnp.dot(h, w_ref[...], preferred_element_type=jnp.float32)


def _mid(parts, b, W, rows_per_block=2000):
    return pl.pallas_call(
        _mid_body,
        grid=(N // rows_per_block,),
        in_specs=[
            pl.BlockSpec((2, rows_per_block, D), lambda i: (0, i, 0)),
            pl.BlockSpec((1, D), lambda i: (0, 0)),
            pl.BlockSpec((D, D), lambda i: (0, 0)),
        ],
        out_specs=pl.BlockSpec((rows_per_block, D), lambda i: (i, 0)),
        out_shape=jax.ShapeDtypeStruct((N, D), jnp.float32),
    )(parts, b, W)


def _fin_body(p_ref, b_ref, x_ref, o_ref):
    h = jnp.maximum(p_ref[0] + p_ref[1] + b_ref[...], 0.0)
    o_ref[...] = (h + x_ref[...]) * 0.5


def _fin(parts, b, x, rows_per_block=2000):
    return pl.pallas_call(
        _fin_body,
        grid=(N // rows_per_block,),
        in_specs=[
            pl.BlockSpec((2, rows_per_block, D), lambda i: (0, i, 0)),
            pl.BlockSpec((1, D), lambda i: (0, 0)),
            pl.BlockSpec((rows_per_block, D), lambda i: (i, 0)),
        ],
        out_specs=pl.BlockSpec((rows_per_block, D), lambda i: (i, 0)),
        out_shape=jax.ShapeDtypeStruct((N, D), jnp.float32),
    )(parts, b, x)


@functools.partial(
    pl.kernel,
    mesh=plsc.VectorSubcoreMesh(core_axis_name="c", subcore_axis_name="s"),
    out_type=jax.ShapeDtypeStruct((2, NROWS, D), jnp.float32),
    scratch_types=[
        pltpu.VMEM((CPT, CHUNK), jnp.int32),
        pltpu.VMEM((CPT, CHUNK), jnp.int32),
        pltpu.VMEM((CHUNK, D), jnp.float32),
        pltpu.VMEM_SHARED((NROWS, D), jnp.float32),
        pltpu.SemaphoreType.DMA,
    ],
)
def _sc_aggregate(src_hbm, dst_hbm, hw_hbm, z_hbm, out_hbm,
                  src_v, dst_v, rows_v, agg_sh, sem):
    cid = lax.axis_index("c")
    sid = lax.axis_index("s")
    wid = cid * NS + sid

    row0 = sid * RPT
    pltpu.sync_copy(z_hbm.at[pl.ds(row0, RPT)], agg_sh.at[pl.ds(row0, RPT)])
    plsc.subcore_barrier()

    pltpu.sync_copy(src_hbm.at[pl.ds(wid * CPT, CPT)], src_v)
    pltpu.sync_copy(dst_hbm.at[pl.ds(wid * CPT, CPT)], dst_v)

    def body(k, carry):
        pltpu.async_copy(hw_hbm.at[src_v.at[k]], rows_v, sem).wait()
        pltpu.sync_copy(rows_v, agg_sh.at[dst_v.at[k]], add=True)
        return carry

    lax.fori_loop(0, CPT, body, 0)

    plsc.subcore_barrier()
    pltpu.sync_copy(agg_sh.at[pl.ds(row0, RPT)],
                    out_hbm.at[cid, pl.ds(row0, RPT)])


def kernel(x, edge_index, W1, b1, W2, b2):
    src = edge_index[0].astype(jnp.int32)
    dst = edge_index[1].astype(jnp.int32)
    pad = E_PAD - E
    src2d = jnp.concatenate([src, jnp.zeros((pad,), jnp.int32)]).reshape(
        NW * CPT, CHUNK)
    dst2d = jnp.concatenate([dst, jnp.full((pad,), N, jnp.int32)]).reshape(
        NW * CPT, CHUNK)
    zeros = jnp.zeros((NROWS, D), jnp.float32)
    b1r = b1.reshape(1, D)
    b2r = b2.reshape(1, D)

    hw1 = _matmul(x, W1)
    parts1 = _sc_aggregate(src2d, dst2d, hw1, zeros)
    hw2 = _mid(parts1, b1r, W2)
    parts2 = _sc_aggregate(src2d, dst2d, hw2, zeros)
    return _fin(parts2, b2r, x)


# speedup vs baseline: 3.4525x; 1.1578x over previous
import functools

import jax
import jax.numpy as jnp
from jax import lax
from jax.experimental import pallas as pl
from jax.experimental.pallas import tpu as pltpu
from jax.experimental.pallas import tpu_sc as plsc

N = 10000
E = 320000
D = 128

NC = 2
NS = 16
NW = NC * NS
CHUNK = 128
CPT = 80
HCPT = 40
E_PAD = NW * CPT * CHUNK
RPT = 632
NROWS = NS * RPT


def _mm_body(h_ref, w_ref, o_ref):
    o_ref[...] = jnp.dot(h_ref[...], w_ref[...],
                         preferred_element_type=jnp.float32)


def _matmul(h, W, rows_per_block=2000):
    return pl.pallas_call(
        _mm_body,
        grid=(N // rows_per_block,),
        in_specs=[
            pl.BlockSpec((rows_per_block, D), lambda i: (i, 0)),
            pl.BlockSpec((D, D), lambda i: (0, 0)),
        ],
        out_specs=pl.BlockSpec((rows_per_block, D), lambda i: (i, 0)),
        out_shape=jax.ShapeDtypeStruct((N, D), jnp.float32),
    )(h, W)


def _mid_body(p_ref, b_ref, w_ref, o_ref):
    h = jnp.maximum(p_ref[0] + p_ref[1] + b_ref[...], 0.0)
    o_ref[...] = jnp.dot(h, w_ref[...], preferred_element_type=jnp.float32)


def _mid(parts, b, W, rows_per_block=2000):
    return pl.pallas_call(
        _mid_body,
        grid=(N // rows_per_block,),
        in_specs=[
            pl.BlockSpec((2, rows_per_block, D), lambda i: (0, i, 0)),
            pl.BlockSpec((1, D), lambda i: (0, 0)),
            pl.BlockSpec((D, D), lambda i: (0, 0)),
        ],
        out_specs=pl.BlockSpec((rows_per_block, D), lambda i: (i, 0)),
        out_shape=jax.ShapeDtypeStruct((N, D), jnp.float32),
    )(parts, b, W)


def _fin_body(p_ref, b_ref, x_ref, o_ref):
    h = jnp.maximum(p_ref[0] + p_ref[1] + b_ref[...], 0.0)
    o_ref[...] = (h + x_ref[...]) * 0.5


def _fin(parts, b, x, rows_per_block=2000):
    return pl.pallas_call(
        _fin_body,
        grid=(N // rows_per_block,),
        in_specs=[
            pl.BlockSpec((2, rows_per_block, D), lambda i: (0, i, 0)),
            pl.BlockSpec((1, D), lambda i: (0, 0)),
            pl.BlockSpec((rows_per_block, D), lambda i: (i, 0)),
        ],
        out_specs=pl.BlockSpec((rows_per_block, D), lambda i: (i, 0)),
        out_shape=jax.ShapeDtypeStruct((N, D), jnp.float32),
    )(parts, b, x)


NBUF = 2
HPAIRS = HCPT // NBUF


@functools.partial(
    pl.kernel,
    mesh=plsc.VectorSubcoreMesh(core_axis_name="c", subcore_axis_name="s"),
    out_type=jax.ShapeDtypeStruct((2, NROWS, D), jnp.float32),
    scratch_types=[
        pltpu.VMEM((HCPT, CHUNK), jnp.int32),
        pltpu.VMEM((HCPT, CHUNK), jnp.int32),
        pltpu.VMEM((CHUNK, D), jnp.float32),
        pltpu.VMEM((CHUNK, D), jnp.float32),
        pltpu.VMEM_SHARED((NROWS, D), jnp.float32),
        pltpu.SemaphoreType.DMA,
        pltpu.SemaphoreType.DMA,
        pltpu.SemaphoreType.DMA,
        pltpu.SemaphoreType.DMA,
    ],
)
def _sc_aggregate(src_hbm, dst_hbm, hw_hbm, z_hbm, out_hbm,
                  src_v, dst_v, rows0, rows1, agg_sh,
                  gs0, gs1, ss0, ss1):
    bufs = (rows0, rows1)
    gsems = (gs0, gs1)
    ssems = (ss0, ss1)
    cid = lax.axis_index("c")
    sid = lax.axis_index("s")
    wid = cid * NS + sid

    row0 = sid * RPT
    pltpu.sync_copy(z_hbm.at[pl.ds(row0, RPT)], agg_sh.at[pl.ds(row0, RPT)])

    for h in range(2):
        pltpu.sync_copy(src_hbm.at[wid, pl.ds(h * HCPT, HCPT)], src_v)
        pltpu.sync_copy(dst_hbm.at[wid, pl.ds(h * HCPT, HCPT)], dst_v)
        if h == 0:
            plsc.subcore_barrier()

        for b in range(NBUF):
            pltpu.async_copy(hw_hbm.at[src_v.at[b]], bufs[b], gsems[b])

        def pair(p, carry):
            for b in range(NBUF):
                k = p * NBUF + b
                pltpu.make_async_copy(hw_hbm.at[src_v.at[k]], bufs[b],
                                      gsems[b]).wait()
                pltpu.async_copy(bufs[b], agg_sh.at[dst_v.at[k]], ssems[b],
                                 add=True)
            for b in range(NBUF):
                kn = p * NBUF + b + NBUF
                pltpu.make_async_copy(bufs[b], agg_sh.at[dst_v.at[0]],
                                      ssems[b]).wait()
                pltpu.async_copy(hw_hbm.at[src_v.at[kn]], bufs[b], gsems[b])
            return carry

        lax.fori_loop(0, HPAIRS - 1, pair, 0)
        for b in range(NBUF):
            k = (HPAIRS - 1) * NBUF + b
            pltpu.make_async_copy(hw_hbm.at[src_v.at[k]], bufs[b],
                                  gsems[b]).wait()
            pltpu.async_copy(bufs[b], agg_sh.at[dst_v.at[k]], ssems[b],
                             add=True)
        for b in range(NBUF):
            pltpu.make_async_copy(bufs[b], agg_sh.at[dst_v.at[0]],
                                  ssems[b]).wait()

    plsc.subcore_barrier()
    pltpu.sync_copy(agg_sh.at[pl.ds(row0, RPT)],
                    out_hbm.at[cid, pl.ds(row0, RPT)])


def kernel(x, edge_index, W1, b1, W2, b2):
    src = edge_index[0].astype(jnp.int32)
    dst = edge_index[1].astype(jnp.int32)
    pad = E_PAD - E
    src3d = jnp.concatenate([src, jnp.zeros((pad,), jnp.int32)]).reshape(
        NW, CPT, CHUNK)
    dst3d = jnp.concatenate([dst, jnp.full((pad,), N, jnp.int32)]).reshape(
        NW, CPT, CHUNK)
    zeros = jnp.zeros((NROWS, D), jnp.float32)
    b1r = b1.reshape(1, D)
    b2r = b2.reshape(1, D)

    hw1 = _matmul(x, W1)
    parts1 = _sc_aggregate(src3d, dst3d, hw1, zeros)
    hw2 = _mid(parts1, b1r, W2)
    parts2 = _sc_aggregate(src3d, dst3d, hw2, zeros)
    return _fin(parts2, b2r, x)
